# baseline (speedup 1.0000x reference)
import functools

import jax
import jax.numpy as jnp
from jax import lax
from jax.experimental import pallas as pl
from jax.experimental.pallas import tpu as pltpu
from jax.experimental.pallas import tpu_sc as plsc

_INFO = plsc.get_sparse_core_info()
_NC, _NS = _INFO.num_cores, _INFO.num_subcores
_NW = _NC * _NS


def _embed_gather(table_hbm, idx_hbm, out_hbm,
                  idx_v0, idx_v1, r128_0, r128_1, r64_0, r64_1,
                  sem_g0, sem_g1, sem_s0, sem_s1,
                  *, b_per_w, chunk):
    wid = lax.axis_index("s") * _NC + lax.axis_index("c")
    base_w = wid * b_per_w
    n_chunks = b_per_w // chunk
    idx_v = (idx_v0, idx_v1)
    r128 = (r128_0, r128_1)
    r64 = (r64_0, r64_1)
    sem_g = (sem_g0, sem_g1)
    sem_s = (sem_s0, sem_s1)

    def chunk_slice(g):
        return pl.ds(pl.multiple_of(base_w + g * chunk, 8), chunk)

    def compact(b):
        def row(i, carry):
            for j in range(4):
                sl = pl.ds(j * 16, 16)
                r64[b][i, sl] = r128[b][i, sl]
            return carry
        lax.fori_loop(0, chunk, row, 0, unroll=4)

    def body(i, carry):
        for b in range(2):
            g = 2 * i + b

            @pl.when(i >= 1)
            def _wait_prev_store():
                pltpu.make_async_copy(
                    r64[b], out_hbm.at[chunk_slice(g), :], sem_s[b]
                ).wait()

            pltpu.sync_copy(idx_hbm.at[chunk_slice(g)], idx_v[b])
            pltpu.async_copy(table_hbm.at[idx_v[b]], r128[b], sem_g[b])
        for b in range(2):
            g = 2 * i + b
            pltpu.make_async_copy(
                table_hbm.at[idx_v[b]], r128[b], sem_g[b]
            ).wait()
            compact(b)
            pltpu.async_copy(r64[b], out_hbm.at[chunk_slice(g), :], sem_s[b])
        return carry

    lax.fori_loop(0, n_chunks // 2, body, 0)
    for b in range(2):
        g = n_chunks - 2 + b
        pltpu.make_async_copy(
            r64[b], out_hbm.at[chunk_slice(g), :], sem_s[b]
        ).wait()


def kernel(input_ids, embed_table):
    B, S = input_ids.shape
    V, D = embed_table.shape
    n = B * S
    assert n % _NW == 0
    b_per_w = n // _NW
    chunk = 160
    assert b_per_w % (2 * chunk) == 0

    idx_flat = input_ids.reshape(n)
    padded = jnp.pad(embed_table, ((0, 0), (0, 128 - D)))

    mesh = plsc.VectorSubcoreMesh(core_axis_name="c", subcore_axis_name="s")
    k = pl.kernel(
        functools.partial(_embed_gather, b_per_w=b_per_w, chunk=chunk),
        mesh=mesh,
        out_type=jax.ShapeDtypeStruct((n, D), jnp.float32),
        scratch_types=[
            pltpu.VMEM((chunk,), jnp.int32),
            pltpu.VMEM((chunk,), jnp.int32),
            pltpu.VMEM((chunk, 128), jnp.float32),
            pltpu.VMEM((chunk, 128), jnp.float32),
            pltpu.VMEM((chunk, D), jnp.float32),
            pltpu.VMEM((chunk, D), jnp.float32),
            pltpu.SemaphoreType.DMA,
            pltpu.SemaphoreType.DMA,
            pltpu.SemaphoreType.DMA,
            pltpu.SemaphoreType.DMA,
        ],
        compiler_params=pltpu.CompilerParams(use_tc_tiling_on_sc=True),
    )
    out = k(padded, idx_flat)
    return out.reshape(B, S, D)
